# baseline (speedup 1.0000x reference)
import jax
import jax.numpy as jnp
from jax.experimental import pallas as pl

B, S, D = 4, 8192, 768
BS = 512


def _imp_kernel(x_ref, o_ref):
    s = pl.program_id(1)
    x = x_ref[0]
    o_ref[0, 0, pl.ds(s * BS, BS)] = jnp.sqrt(jnp.sum(x * x, axis=-1))


def kernel(tokens):
    imp = pl.pallas_call(
        _imp_kernel,
        grid=(B, S // BS),
        in_specs=[pl.BlockSpec((1, BS, D), lambda b, s: (b, s, 0))],
        out_specs=pl.BlockSpec((1, 1, S), lambda b, s: (b, 0, 0)),
        out_shape=jax.ShapeDtypeStruct((B, 1, S), jnp.float32),
    )(tokens)
    imp = imp.reshape(B, S)
    k = S // 2
    _, keep_idx = jax.lax.top_k(imp, k)
    keep_mask = jnp.zeros((B, S), jnp.bool_).at[
        jnp.arange(B)[:, None], keep_idx
    ].set(True)
    sorted_idx = jnp.sort(keep_idx, axis=1)
    pruned = jnp.take_along_axis(tokens, sorted_idx[:, :, None], axis=1)
    return (pruned, keep_mask)
